# Initial kernel scaffold; baseline (speedup 1.0000x reference)
import jax
import jax.numpy as jnp
from jax.experimental import pallas as pl


def kernel(cat_00, cat_01, cat_02, cat_03, cat_04, cat_05, cat_06, cat_07, cat_08, cat_09, cat_10, cat_11, cat_12, cat_13, cat_14, cat_15, cat_16, cat_17, cat_18, cat_19, cat_20, cat_21, cat_22, cat_23, cat_24, cat_25, num, tables, numeric_kernel, bias):
    raise NotImplementedError("write your pallas kernel here")


# speedup vs baseline: 1.3178x; 1.3178x over previous
import functools

import jax
import jax.numpy as jnp
from jax import lax
from jax.experimental import pallas as pl
from jax.experimental.pallas import tpu as pltpu
from jax.experimental.pallas import tpu_sc as plsc

_N_FIELDS = 26
_VOCAB = 100000
_BATCH = 16384
_NUM_DIM = 13

_NC, _NS, _L = 2, 16, 16
_NW = _NC * _NS
_BPW = _BATCH // _NW
_G = _BPW // _L
_ND1 = _NUM_DIM + 1


def _sc_body(table_ref, idx_ref, numt_ref, nkb_ref, out_ref,
             idx_v, rows_v, numt_v, nkb_v, out_v, sem):
    c = lax.axis_index("c")
    s = lax.axis_index("s")
    wid = s * _NC + c

    pltpu.sync_copy(idx_ref.at[wid], idx_v)
    pltpu.sync_copy(numt_ref.at[wid], numt_v)
    pltpu.sync_copy(nkb_ref, nkb_v)

    def _field_off(i, _):
        off = i * _VOCAB
        def _grp(g, _):
            sl = pl.ds(i * _BPW + g * _L, _L)
            idx_v[sl] = idx_v[sl] + off
            return 0
        lax.fori_loop(0, _G, _grp, 0)
        return 0
    lax.fori_loop(0, _N_FIELDS, _field_off, 0)

    pltpu.async_copy(table_ref.at[idx_v], rows_v, sem).wait()

    nkb_vec = nkb_v[:]

    def _gbody(g, _):
        sl = pl.ds(g * _L, _L)
        acc = jnp.zeros((_L,), jnp.float32)
        for d in range(_ND1):
            acc = acc + nkb_vec[d] * numt_v[d, sl]
        def _fbody(i, a):
            return a + rows_v[pl.ds(i * _BPW + g * _L, _L)]
        acc = lax.fori_loop(0, _N_FIELDS, _fbody, acc)
        out_v[sl] = acc
        return 0
    lax.fori_loop(0, _G, _gbody, 0)

    pltpu.sync_copy(out_v, out_ref.at[wid])


@functools.partial(
    pl.kernel,
    out_type=jax.ShapeDtypeStruct((_NW, _BPW), jnp.float32),
    mesh=plsc.VectorSubcoreMesh(core_axis_name="c", subcore_axis_name="s",
                                num_cores=_NC, num_subcores=_NS),
    scratch_types=[
        pltpu.VMEM((_N_FIELDS * _BPW,), jnp.int32),
        pltpu.VMEM((_N_FIELDS * _BPW,), jnp.float32),
        pltpu.VMEM((_ND1, _BPW), jnp.float32),
        pltpu.VMEM((_L,), jnp.float32),
        pltpu.VMEM((_BPW,), jnp.float32),
        pltpu.SemaphoreType.DMA,
    ],
)
def _sc_linear_features(table_ref, idx_ref, numt_ref, nkb_ref, out_ref,
                        idx_v, rows_v, numt_v, nkb_v, out_v, sem):
    _sc_body(table_ref, idx_ref, numt_ref, nkb_ref, out_ref,
             idx_v, rows_v, numt_v, nkb_v, out_v, sem)


@jax.jit
def _run(cats, num, tables, numeric_kernel, bias):
    idx = jnp.concatenate(cats, axis=1).T
    idx = idx.reshape(_N_FIELDS, _NW, _BPW).transpose(1, 0, 2)
    idx = idx.reshape(_NW, _N_FIELDS * _BPW)
    numt = jnp.concatenate(
        [num.T, jnp.ones((1, _BATCH), jnp.float32)], axis=0)
    numt = numt.reshape(_ND1, _NW, _BPW).transpose(1, 0, 2)
    nkb = jnp.concatenate(
        [numeric_kernel[:, 0], bias,
         jnp.zeros((_L - _ND1,), jnp.float32)])
    table_flat = tables.reshape(_N_FIELDS * _VOCAB)
    out = _sc_linear_features(table_flat, idx, numt, nkb)
    return out.reshape(_BATCH, 1)


def kernel(cat_00, cat_01, cat_02, cat_03, cat_04, cat_05, cat_06, cat_07,
           cat_08, cat_09, cat_10, cat_11, cat_12, cat_13, cat_14, cat_15,
           cat_16, cat_17, cat_18, cat_19, cat_20, cat_21, cat_22, cat_23,
           cat_24, cat_25, num, tables, numeric_kernel, bias):
    cats = (cat_00, cat_01, cat_02, cat_03, cat_04, cat_05, cat_06, cat_07,
            cat_08, cat_09, cat_10, cat_11, cat_12, cat_13, cat_14, cat_15,
            cat_16, cat_17, cat_18, cat_19, cat_20, cat_21, cat_22, cat_23,
            cat_24, cat_25)
    return _run(cats, num, tables, numeric_kernel, bias)


# speedup vs baseline: 2.9949x; 2.2726x over previous
import functools

import jax
import jax.numpy as jnp
from jax import lax
from jax.experimental import pallas as pl
from jax.experimental.pallas import tpu as pltpu
from jax.experimental.pallas import tpu_sc as plsc

_N_FIELDS = 26
_VOCAB = 100000
_BATCH = 16384
_NUM_DIM = 13

_VPAD = 100096
_NC, _NS, _L = 2, 16, 16
_NW = _NC * _NS
_BPW = _BATCH // _NW
_G = _BPW // _L


def _sc_body(table_ref, idx_ref, numt_ref, nkb_ref, out_ref,
             idx_v, rows_v, numt_v, nkb_v, out_v, sem):
    c = lax.axis_index("c")
    s = lax.axis_index("s")
    wid = s * _NC + c
    base = wid * _BPW

    pltpu.sync_copy(idx_ref.at[wid], idx_v)
    pltpu.sync_copy(numt_ref.at[:, pl.ds(base, _BPW)], numt_v)
    pltpu.sync_copy(nkb_ref, nkb_v)

    def _field_off(i, _):
        off = i * _VPAD
        def _grp(g, _):
            sl = pl.ds(i * _BPW + g * _L, _L)
            idx_v[sl] = idx_v[sl] + off
            return 0
        lax.fori_loop(0, _G, _grp, 0)
        return 0
    lax.fori_loop(0, _N_FIELDS, _field_off, 0)

    pltpu.async_copy(table_ref.at[idx_v], rows_v, sem).wait()

    nkb_vec = nkb_v[:]

    def _gbody(g, _):
        sl = pl.ds(g * _L, _L)
        acc = jnp.broadcast_to(nkb_vec[_NUM_DIM], (_L,))
        for d in range(_NUM_DIM):
            acc = acc + nkb_vec[d] * numt_v[d, sl]
        def _fbody(i, a):
            return a + rows_v[pl.ds(i * _BPW + g * _L, _L)]
        acc = lax.fori_loop(0, _N_FIELDS, _fbody, acc)
        out_v[sl] = acc
        return 0
    lax.fori_loop(0, _G, _gbody, 0)

    pltpu.sync_copy(out_v, out_ref.at[wid])


@functools.partial(
    pl.kernel,
    out_type=jax.ShapeDtypeStruct((_NW, _BPW), jnp.float32),
    mesh=plsc.VectorSubcoreMesh(core_axis_name="c", subcore_axis_name="s",
                                num_cores=_NC, num_subcores=_NS),
    scratch_types=[
        pltpu.VMEM((_N_FIELDS * _BPW,), jnp.int32),
        pltpu.VMEM((_N_FIELDS * _BPW,), jnp.float32),
        pltpu.VMEM((_NUM_DIM, _BPW), jnp.float32),
        pltpu.VMEM((_L,), jnp.float32),
        pltpu.VMEM((_BPW,), jnp.float32),
        pltpu.SemaphoreType.DMA,
    ],
)
def _sc_linear_features(table_ref, idx_ref, numt_ref, nkb_ref, out_ref,
                        idx_v, rows_v, numt_v, nkb_v, out_v, sem):
    _sc_body(table_ref, idx_ref, numt_ref, nkb_ref, out_ref,
             idx_v, rows_v, numt_v, nkb_v, out_v, sem)


@jax.jit
def _run(cats, num, tables, numeric_kernel, bias):
    idx = jnp.concatenate(cats, axis=1).T
    idx = idx.reshape(_N_FIELDS, _NW, _BPW).transpose(1, 0, 2)
    idx = idx.reshape(_NW, _N_FIELDS * _BPW)
    numt = num.T
    nkb = jnp.concatenate(
        [numeric_kernel[:, 0], bias,
         jnp.zeros((_L - _NUM_DIM - 1,), jnp.float32)])
    table_flat = jnp.pad(tables.reshape(_N_FIELDS, _VOCAB),
                         ((0, 0), (0, _VPAD - _VOCAB)))
    table_flat = table_flat.reshape(_N_FIELDS * _VPAD)
    out = _sc_linear_features(table_flat, idx, numt, nkb)
    return out.reshape(_BATCH, 1)


def kernel(cat_00, cat_01, cat_02, cat_03, cat_04, cat_05, cat_06, cat_07,
           cat_08, cat_09, cat_10, cat_11, cat_12, cat_13, cat_14, cat_15,
           cat_16, cat_17, cat_18, cat_19, cat_20, cat_21, cat_22, cat_23,
           cat_24, cat_25, num, tables, numeric_kernel, bias):
    cats = (cat_00, cat_01, cat_02, cat_03, cat_04, cat_05, cat_06, cat_07,
            cat_08, cat_09, cat_10, cat_11, cat_12, cat_13, cat_14, cat_15,
            cat_16, cat_17, cat_18, cat_19, cat_20, cat_21, cat_22, cat_23,
            cat_24, cat_25)
    return _run(cats, num, tables, numeric_kernel, bias)


# speedup vs baseline: 3.1544x; 1.0533x over previous
import functools

import jax
import jax.numpy as jnp
from jax import lax
from jax.experimental import pallas as pl
from jax.experimental.pallas import tpu as pltpu
from jax.experimental.pallas import tpu_sc as plsc

_N_FIELDS = 26
_VOCAB = 100000
_BATCH = 16384
_NUM_DIM = 13

_VPAD = 100096
_NC, _NS, _L = 2, 16, 16
_NW = _NC * _NS
_BPW = _BATCH // _NW
_G = _BPW // _L


def _sc_body(table_ref, idx_ref, numt_ref, nkb_ref, out_ref,
             idx_v, rows_v, numt_v, nkb_v, out_v, sem):
    c = lax.axis_index("c")
    s = lax.axis_index("s")
    wid = s * _NC + c
    base = wid * _BPW

    pltpu.sync_copy(idx_ref.at[wid], idx_v)
    pltpu.sync_copy(numt_ref.at[:, pl.ds(base, _BPW)], numt_v)
    pltpu.sync_copy(nkb_ref, nkb_v)

    pltpu.async_copy(table_ref.at[idx_v], rows_v, sem).wait()

    nkb_vec = nkb_v[:]

    def _gbody(g, _):
        sl = pl.ds(g * _L, _L)
        acc = jnp.broadcast_to(nkb_vec[_NUM_DIM], (_L,))
        for d in range(_NUM_DIM):
            acc = acc + nkb_vec[d] * numt_v[d, sl]
        for i in range(_N_FIELDS):
            acc = acc + rows_v[pl.ds(i * _BPW + g * _L, _L)]
        out_v[sl] = acc
        return 0
    lax.fori_loop(0, _G, _gbody, 0)

    pltpu.sync_copy(out_v, out_ref.at[wid])


@functools.partial(
    pl.kernel,
    out_type=jax.ShapeDtypeStruct((_NW, _BPW), jnp.float32),
    mesh=plsc.VectorSubcoreMesh(core_axis_name="c", subcore_axis_name="s",
                                num_cores=_NC, num_subcores=_NS),
    scratch_types=[
        pltpu.VMEM((_N_FIELDS * _BPW,), jnp.int32),
        pltpu.VMEM((_N_FIELDS * _BPW,), jnp.float32),
        pltpu.VMEM((_NUM_DIM, _BPW), jnp.float32),
        pltpu.VMEM((_L,), jnp.float32),
        pltpu.VMEM((_BPW,), jnp.float32),
        pltpu.SemaphoreType.DMA,
    ],
)
def _sc_linear_features(table_ref, idx_ref, numt_ref, nkb_ref, out_ref,
                        idx_v, rows_v, numt_v, nkb_v, out_v, sem):
    _sc_body(table_ref, idx_ref, numt_ref, nkb_ref, out_ref,
             idx_v, rows_v, numt_v, nkb_v, out_v, sem)


@jax.jit
def _run(cats, num, tables, numeric_kernel, bias):
    off = (jnp.arange(_N_FIELDS, dtype=jnp.int32) * _VPAD)[:, None]
    idx = jnp.concatenate(cats, axis=1).T + off
    idx = idx.reshape(_N_FIELDS, _NW, _BPW).transpose(1, 0, 2)
    idx = idx.reshape(_NW, _N_FIELDS * _BPW)
    numt = num.T
    nkb = jnp.concatenate(
        [numeric_kernel[:, 0], bias,
         jnp.zeros((_L - _NUM_DIM - 1,), jnp.float32)])
    table_flat = jnp.pad(tables.reshape(_N_FIELDS, _VOCAB),
                         ((0, 0), (0, _VPAD - _VOCAB)))
    table_flat = table_flat.reshape(_N_FIELDS * _VPAD)
    out = _sc_linear_features(table_flat, idx, numt, nkb)
    return out.reshape(_BATCH, 1)


def kernel(cat_00, cat_01, cat_02, cat_03, cat_04, cat_05, cat_06, cat_07,
           cat_08, cat_09, cat_10, cat_11, cat_12, cat_13, cat_14, cat_15,
           cat_16, cat_17, cat_18, cat_19, cat_20, cat_21, cat_22, cat_23,
           cat_24, cat_25, num, tables, numeric_kernel, bias):
    cats = (cat_00, cat_01, cat_02, cat_03, cat_04, cat_05, cat_06, cat_07,
            cat_08, cat_09, cat_10, cat_11, cat_12, cat_13, cat_14, cat_15,
            cat_16, cat_17, cat_18, cat_19, cat_20, cat_21, cat_22, cat_23,
            cat_24, cat_25)
    return _run(cats, num, tables, numeric_kernel, bias)


# speedup vs baseline: 4.3735x; 1.3865x over previous
import functools

import jax
import jax.numpy as jnp
from jax import lax
from jax.experimental import pallas as pl
from jax.experimental.pallas import tpu as pltpu
from jax.experimental.pallas import tpu_sc as plsc

_N_FIELDS = 26
_VOCAB = 100000
_BATCH = 16384
_NUM_DIM = 13

_VPAD = 100096
_NC, _NS, _L = 2, 16, 16
_NW = _NC * _NS
_BPW = _BATCH // _NW
_G = _BPW // _L


def _sc_body(table_ref, idx_ref, numt_ref, nkb_ref, out_ref,
             idx_v, rows_v, numt_v, nkb_v, out_v, sem):
    c = lax.axis_index("c")
    s = lax.axis_index("s")
    wid = s * _NC + c
    base = wid * _BPW

    pltpu.sync_copy(idx_ref.at[wid], idx_v)
    pltpu.sync_copy(numt_ref.at[:, pl.ds(base, _BPW)], numt_v)
    pltpu.sync_copy(nkb_ref, nkb_v)

    pltpu.async_copy(table_ref.at[idx_v], rows_v, sem).wait()

    nkb_vec = nkb_v[:]

    def _gbody(g, _):
        sl = pl.ds(g * _L, _L)
        acc = jnp.broadcast_to(nkb_vec[_NUM_DIM], (_L,))
        for d in range(_NUM_DIM):
            acc = acc + nkb_vec[d] * numt_v[d, sl]
        for i in range(_N_FIELDS):
            acc = acc + rows_v[pl.ds(i * _BPW + g * _L, _L)]
        out_v[sl] = acc
        return 0
    lax.fori_loop(0, _G, _gbody, 0)

    pltpu.sync_copy(out_v, out_ref.at[wid])


@functools.partial(
    pl.kernel,
    out_type=jax.ShapeDtypeStruct((_N_FIELDS * _VOCAB,), jnp.float32),
    mesh=plsc.VectorSubcoreMesh(core_axis_name="c", subcore_axis_name="s",
                                num_cores=_NC, num_subcores=_NS),
    scratch_types=[
        pltpu.VMEM((_VOCAB,), jnp.float32),
    ],
)
def _sc_flatten_tables(tv_ref, flat_ref, buf_v):
    c = lax.axis_index("c")
    s = lax.axis_index("s")
    wid = s * _NC + c

    @pl.when(wid < _N_FIELDS)
    def _():
        pltpu.sync_copy(tv_ref.at[wid, 0], buf_v)
        pltpu.sync_copy(buf_v, flat_ref.at[pl.ds(wid * _VOCAB, _VOCAB)])


@functools.partial(
    pl.kernel,
    out_type=jax.ShapeDtypeStruct((_NW, _BPW), jnp.float32),
    mesh=plsc.VectorSubcoreMesh(core_axis_name="c", subcore_axis_name="s",
                                num_cores=_NC, num_subcores=_NS),
    scratch_types=[
        pltpu.VMEM((_N_FIELDS * _BPW,), jnp.int32),
        pltpu.VMEM((_N_FIELDS * _BPW,), jnp.float32),
        pltpu.VMEM((_NUM_DIM, _BPW), jnp.float32),
        pltpu.VMEM((_L,), jnp.float32),
        pltpu.VMEM((_BPW,), jnp.float32),
        pltpu.SemaphoreType.DMA,
    ],
)
def _sc_linear_features(table_ref, idx_ref, numt_ref, nkb_ref, out_ref,
                        idx_v, rows_v, numt_v, nkb_v, out_v, sem):
    _sc_body(table_ref, idx_ref, numt_ref, nkb_ref, out_ref,
             idx_v, rows_v, numt_v, nkb_v, out_v, sem)


@jax.jit
def _run(cats, num, tables, numeric_kernel, bias):
    off = (jnp.arange(_N_FIELDS, dtype=jnp.int32) * _VOCAB)[:, None]
    idx = jnp.concatenate(cats, axis=1).T + off
    idx = idx.reshape(_N_FIELDS, _NW, _BPW).transpose(1, 0, 2)
    idx = idx.reshape(_NW, _N_FIELDS * _BPW)
    numt = num.T
    nkb = jnp.concatenate(
        [numeric_kernel[:, 0], bias,
         jnp.zeros((_L - _NUM_DIM - 1,), jnp.float32)])
    table_flat = _sc_flatten_tables(jnp.transpose(tables, (0, 2, 1)))
    out = _sc_linear_features(table_flat, idx, numt, nkb)
    return out.reshape(_BATCH, 1)


def kernel(cat_00, cat_01, cat_02, cat_03, cat_04, cat_05, cat_06, cat_07,
           cat_08, cat_09, cat_10, cat_11, cat_12, cat_13, cat_14, cat_15,
           cat_16, cat_17, cat_18, cat_19, cat_20, cat_21, cat_22, cat_23,
           cat_24, cat_25, num, tables, numeric_kernel, bias):
    cats = (cat_00, cat_01, cat_02, cat_03, cat_04, cat_05, cat_06, cat_07,
            cat_08, cat_09, cat_10, cat_11, cat_12, cat_13, cat_14, cat_15,
            cat_16, cat_17, cat_18, cat_19, cat_20, cat_21, cat_22, cat_23,
            cat_24, cat_25)
    return _run(cats, num, tables, numeric_kernel, bias)
